# baseline (speedup 1.0000x reference)
import functools

import jax
import jax.numpy as jnp
from jax import lax
from jax.experimental import pallas as pl
from jax.experimental.pallas import tpu as pltpu
from jax.experimental.pallas import tpu_sc as plsc

N = 10000
E = 320000
DIN = 128
H = 64
G = 16
GC = 3

NC = 2
NS = 16
CHUNK = 128
GRP = 4
CPT0 = 160
CPT1 = 0
E_PAD = NS * (CPT0 + CPT1) * CHUNK
RPT = 632
N_PAD = NS * RPT
DEG_W = 16384

_DOT = dict(precision=lax.Precision.HIGHEST, preferred_element_type=jnp.float32)


def _sc_edge_body(with_deg, *refs):
  if with_deg:
    (y_hbm, src_hbm, dst_hbm, acc_out, deg_out, src_v, dst_v,
     r0, r1, r2, r3, onesw_v, zw_v, acc_sh, deg_sh,
     g0, g1, g2, g3, s0, s1, s2, s3, dsem) = refs
  else:
    (y_hbm, src_hbm, dst_hbm, acc_out, src_v, dst_v,
     r0, r1, r2, r3, acc_sh,
     g0, g1, g2, g3, s0, s1, s2, s3) = refs
  rows = [r0, r1, r2, r3]
  gsem = [g0, g1, g2, g3]
  ssem = [s0, s1, s2, s3]
  c = lax.axis_index("c")
  s = lax.axis_index("s")
  cbase = jnp.where(c == 0, s * CPT0, NS * CPT0 + s * CPT1)
  ngrp = jnp.where(c == 0, CPT0 // (2 * GRP), CPT1 // (2 * GRP))
  zeros16 = jnp.zeros((16,), jnp.float32)
  ones16 = jnp.ones((16,), jnp.float32)

  def _z(i, _):
    r0[i >> 2, pl.ds((i & 3) * 16, 16)] = zeros16
    return 0
  lax.fori_loop(0, CHUNK * 4, _z, 0)
  for k in range(RPT // CHUNK):
    pltpu.sync_copy(r0, acc_sh.at[pl.ds(s * RPT + k * CHUNK, CHUNK)])
  rem = RPT - (RPT // CHUNK) * CHUNK
  pltpu.sync_copy(r0.at[pl.ds(0, rem)],
                  acc_sh.at[pl.ds(s * RPT + RPT - rem, rem)])
  if with_deg:
    for g in range(CHUNK // 16):
      onesw_v[pl.ds(g * 16, 16)] = ones16
    def _zd(i, _):
      zw_v[pl.ds(i * 16, 16)] = zeros16
      return 0
    lax.fori_loop(0, DEG_W // NS // 16, _zd, 0)
    pltpu.sync_copy(zw_v, deg_sh.at[pl.ds(s * (DEG_W // NS), DEG_W // NS)])
  plsc.subcore_barrier()

  def _super(u, _):
    pltpu.sync_copy(src_hbm.at[pl.ds(cbase + u * 2 * GRP, 2 * GRP)], src_v)
    pltpu.sync_copy(dst_hbm.at[pl.ds(cbase + u * 2 * GRP, 2 * GRP)], dst_v)
    for half in (0, GRP):
      gd = [pltpu.async_copy(y_hbm.at[src_v.at[half + b]], rows[b], gsem[b])
            for b in range(GRP)]
      sd = []
      dd = []
      for b in range(GRP):
        gd[b].wait()
        sd.append(pltpu.async_copy(rows[b], acc_sh.at[dst_v.at[half + b]],
                                   ssem[b], add=True))
        if with_deg:
          dd.append(pltpu.async_copy(onesw_v, deg_sh.at[dst_v.at[half + b]],
                                     dsem, add=True))
      for d in sd + dd:
        d.wait()
    return 0
  lax.fori_loop(0, ngrp, _super, 0)
  plsc.subcore_barrier()

  off = c * N_PAD + s * RPT
  pltpu.sync_copy(acc_sh.at[pl.ds(s * RPT, RPT)], acc_out.at[pl.ds(off, RPT)])
  if with_deg:
    w = DEG_W // NS
    pltpu.sync_copy(deg_sh.at[pl.ds(s * w, w)],
                    deg_out.at[pl.ds(c * DEG_W + s * w, w)])


def _make_sc_kernel(with_deg):
  mesh = plsc.VectorSubcoreMesh(core_axis_name="c", subcore_axis_name="s")
  out_type = [jax.ShapeDtypeStruct((NC * N_PAD, H), jnp.float32)]
  scratch = [
      pltpu.VMEM((2 * GRP, CHUNK), jnp.int32),
      pltpu.VMEM((2 * GRP, CHUNK), jnp.int32),
  ]
  scratch += [pltpu.VMEM((CHUNK, H), jnp.float32) for _ in range(GRP)]
  if with_deg:
    out_type.append(jax.ShapeDtypeStruct((NC * DEG_W,), jnp.float32))
    scratch += [
        pltpu.VMEM((CHUNK,), jnp.float32),
        pltpu.VMEM((DEG_W // NS,), jnp.float32),
    ]
  scratch.append(pltpu.VMEM_SHARED((N_PAD, H), jnp.float32))
  if with_deg:
    scratch.append(pltpu.VMEM_SHARED((DEG_W,), jnp.float32))
  scratch += [pltpu.SemaphoreType.DMA for _ in range(2 * GRP)]
  if with_deg:
    scratch.append(pltpu.SemaphoreType.DMA)
  return pl.kernel(
      functools.partial(_sc_edge_body, with_deg),
      out_type=out_type, mesh=mesh, scratch_types=scratch,
      compiler_params=pltpu.CompilerParams(needs_layout_passes=False,
                                          use_tc_tiling_on_sc=False),
      name="sage_edge_agg_deg" if with_deg else "sage_edge_agg")


_sc_agg_deg = _make_sc_kernel(True)
_sc_agg = _make_sc_kernel(False)


BN = 1024


def _inv_deg(deg_ref):
  d = deg_ref[...]
  return 1.0 / jnp.maximum(d[0] + d[1], 1.0)


def _combine(acc_ref, deg_ref, y_ref):
  a = acc_ref[...]
  agg = a[0] + a[1]
  return jnp.maximum(agg * _inv_deg(deg_ref) + y_ref[...][:, H:], 0.0)


def _tc_pre_body(x_ref, w_ref, b_ref, y_ref):
  y_ref[...] = jnp.dot(x_ref[...], w_ref[...], **_DOT) + b_ref[...]


def _tc_mid_body(acc_ref, deg_ref, y_ref, w_ref, b_ref, yout_ref):
  h = _combine(acc_ref, deg_ref, y_ref)
  yout_ref[...] = jnp.dot(h, w_ref[...], **_DOT) + b_ref[...]


def _tc_fin_body(acc_ref, deg_ref, y_ref, lng_ref, lnb_ref, batch_ref,
                 u_ref, wpu_ref, wph_ref, bp_ref, wo_ref, bo_ref, out_ref):
  h = _combine(acc_ref, deg_ref, y_ref)
  mu = jnp.mean(h, axis=-1, keepdims=True)
  hc = h - mu
  var = jnp.mean(hc * hc, axis=-1, keepdims=True)
  hn = hc * lax.rsqrt(var + 1e-5) * lng_ref[...] + lnb_ref[...]
  gid = lax.broadcasted_iota(jnp.int32, (BN, G), 1)
  oneh = (batch_ref[...] == gid).astype(jnp.float32)
  up = jnp.dot(u_ref[...], wpu_ref[...], **_DOT)
  z = jnp.maximum(
      jnp.dot(hn, wph_ref[...], **_DOT) + jnp.dot(oneh, up, **_DOT)
      + bp_ref[...], 0.0)
  out_ref[...] = jnp.dot(z, wo_ref[...], **_DOT) + bo_ref[...]


def _row_spec(width):
  return pl.BlockSpec((BN, width), lambda i: (i, 0))


def _full_spec(shape):
  return pl.BlockSpec(shape, lambda i: tuple(0 for _ in shape))


def _acc_specs():
  return [pl.BlockSpec((2, BN, H), lambda i: (0, i, 0)),
          pl.BlockSpec((2, BN, 1), lambda i: (0, i, 0))]


_GRID = (10,)

_tc_pre = pl.pallas_call(
    _tc_pre_body,
    grid=_GRID,
    in_specs=[_row_spec(DIN), _full_spec((DIN, 2 * H)), _full_spec((1, 2 * H))],
    out_specs=_row_spec(2 * H),
    out_shape=jax.ShapeDtypeStruct((N, 2 * H), jnp.float32),
)

_tc_mid = pl.pallas_call(
    _tc_mid_body,
    grid=_GRID,
    in_specs=_acc_specs() + [_row_spec(2 * H), _full_spec((H, 2 * H)),
                             _full_spec((1, 2 * H))],
    out_specs=_row_spec(2 * H),
    out_shape=jax.ShapeDtypeStruct((N, 2 * H), jnp.float32),
)

_tc_fin = pl.pallas_call(
    _tc_fin_body,
    grid=_GRID,
    in_specs=_acc_specs() + [
        _row_spec(2 * H), _full_spec((1, H)), _full_spec((1, H)),
        _row_spec(1),
        _full_spec((G, 8)), _full_spec((8, H)),
        _full_spec((H, H)), _full_spec((1, H)),
        _full_spec((H, 2)), _full_spec((1, 2)),
    ],
    out_specs=_row_spec(2),
    out_shape=jax.ShapeDtypeStruct((N, 2), jnp.float32),
)


def kernel(x, edge_index, batch, u, Wl0, bl0, Wr0, Wl1, bl1, Wr1,
           Wl2, bl2, Wr2, ln_g, ln_b, Wp, bp, Wo, bo):
  pad = E_PAD - E
  src = jnp.concatenate([edge_index[0] * 2, jnp.zeros((pad,), jnp.int32)])
  dst = jnp.concatenate([edge_index[1], jnp.full((pad,), N, jnp.int32)])
  src2 = src.reshape(E_PAD // CHUNK, CHUNK)
  dst2 = dst.reshape(E_PAD // CHUNK, CHUNK)

  zH = jnp.zeros((H,), jnp.float32)
  w0 = jnp.concatenate([Wl0.T, Wr0.T], axis=1)
  w1 = jnp.concatenate([Wl1.T, Wr1.T], axis=1)
  w2 = jnp.concatenate([Wl2.T, Wr2.T], axis=1)
  b0 = jnp.concatenate([zH, bl0]).reshape(1, 2 * H)
  b1 = jnp.concatenate([zH, bl1]).reshape(1, 2 * H)
  b2 = jnp.concatenate([zH, bl2]).reshape(1, 2 * H)
  u_p = jnp.pad(u, ((0, 0), (0, 8 - GC)))
  wpu = jnp.pad(Wp[:, H:].T, ((0, 8 - GC), (0, 0)))
  wph = Wp[:, :H].T
  row = lambda v: v.reshape(1, -1)
  batch2 = batch.reshape(N, 1)

  y0 = _tc_pre(x, w0, b0)
  acc0, deg = _sc_agg_deg(y0.reshape(2 * N, H), src2, dst2)
  acc0 = acc0.reshape(NC, N_PAD, H)
  deg = deg.reshape(NC, DEG_W, 1)

  y1 = _tc_mid(acc0, deg, y0, w1, b1)
  acc1 = _sc_agg(y1.reshape(2 * N, H), src2, dst2)[0].reshape(NC, N_PAD, H)
  y2 = _tc_mid(acc1, deg, y1, w2, b2)
  acc2 = _sc_agg(y2.reshape(2 * N, H), src2, dst2)[0].reshape(NC, N_PAD, H)

  return _tc_fin(acc2, deg, y2, row(ln_g), row(ln_b), batch2,
                 u_p, wpu, wph, row(bp), Wo.T, row(bo))


# speedup vs baseline: 1.1905x; 1.1905x over previous
import functools

import jax
import jax.numpy as jnp
from jax import lax
from jax.experimental import pallas as pl
from jax.experimental.pallas import tpu as pltpu
from jax.experimental.pallas import tpu_sc as plsc

N = 10000
E = 320000
DIN = 128
H = 64
G = 16
GC = 3

NC = 2
NS = 16
CHUNK = 128
GRP = 4
CPT0 = 96
CPT1 = 64
E_PAD = NS * (CPT0 + CPT1) * CHUNK
RPT = 632
N_PAD = NS * RPT
DEG_W = 16384

_DOT = dict(precision=lax.Precision.HIGHEST, preferred_element_type=jnp.float32)


def _sc_edge_body(with_deg, *refs):
  if with_deg:
    (y_hbm, src_hbm, dst_hbm, acc_out, deg_out, src_v, dst_v,
     r0, r1, r2, r3, onesw_v, zw_v, acc_sh, deg_sh,
     g0, g1, g2, g3, s0, s1, s2, s3, dsem) = refs
  else:
    (y_hbm, src_hbm, dst_hbm, acc_out, src_v, dst_v,
     r0, r1, r2, r3, acc_sh,
     g0, g1, g2, g3, s0, s1, s2, s3) = refs
  rows = [r0, r1, r2, r3]
  gsem = [g0, g1, g2, g3]
  ssem = [s0, s1, s2, s3]
  c = lax.axis_index("c")
  s = lax.axis_index("s")
  cbase = jnp.where(c == 0, s * CPT0, NS * CPT0 + s * CPT1)
  ngrp = jnp.where(c == 0, CPT0 // (2 * GRP), CPT1 // (2 * GRP))
  zeros16 = jnp.zeros((16,), jnp.float32)
  ones16 = jnp.ones((16,), jnp.float32)

  def _z(i, _):
    r0[i >> 2, pl.ds((i & 3) * 16, 16)] = zeros16
    return 0
  lax.fori_loop(0, CHUNK * 4, _z, 0)
  for k in range(RPT // CHUNK):
    pltpu.sync_copy(r0, acc_sh.at[pl.ds(s * RPT + k * CHUNK, CHUNK)])
  rem = RPT - (RPT // CHUNK) * CHUNK
  pltpu.sync_copy(r0.at[pl.ds(0, rem)],
                  acc_sh.at[pl.ds(s * RPT + RPT - rem, rem)])
  if with_deg:
    for g in range(CHUNK // 16):
      onesw_v[pl.ds(g * 16, 16)] = ones16
    def _zd(i, _):
      zw_v[pl.ds(i * 16, 16)] = zeros16
      return 0
    lax.fori_loop(0, DEG_W // NS // 16, _zd, 0)
    pltpu.sync_copy(zw_v, deg_sh.at[pl.ds(s * (DEG_W // NS), DEG_W // NS)])
  plsc.subcore_barrier()

  def _super(u, _):
    pltpu.sync_copy(src_hbm.at[pl.ds(cbase + u * 2 * GRP, 2 * GRP)], src_v)
    pltpu.sync_copy(dst_hbm.at[pl.ds(cbase + u * 2 * GRP, 2 * GRP)], dst_v)
    for half in (0, GRP):
      gd = [pltpu.async_copy(y_hbm.at[src_v.at[half + b]], rows[b], gsem[b])
            for b in range(GRP)]
      sd = []
      dd = []
      for b in range(GRP):
        gd[b].wait()
        sd.append(pltpu.async_copy(rows[b], acc_sh.at[dst_v.at[half + b]],
                                   ssem[b], add=True))
        if with_deg:
          dd.append(pltpu.async_copy(onesw_v, deg_sh.at[dst_v.at[half + b]],
                                     dsem, add=True))
      for d in sd + dd:
        d.wait()
    return 0
  lax.fori_loop(0, ngrp, _super, 0)
  plsc.subcore_barrier()

  off = c * N_PAD + s * RPT
  pltpu.sync_copy(acc_sh.at[pl.ds(s * RPT, RPT)], acc_out.at[pl.ds(off, RPT)])
  if with_deg:
    w = DEG_W // NS
    pltpu.sync_copy(deg_sh.at[pl.ds(s * w, w)],
                    deg_out.at[pl.ds(c * DEG_W + s * w, w)])


def _make_sc_kernel(with_deg):
  mesh = plsc.VectorSubcoreMesh(core_axis_name="c", subcore_axis_name="s")
  out_type = [jax.ShapeDtypeStruct((NC * N_PAD, H), jnp.float32)]
  scratch = [
      pltpu.VMEM((2 * GRP, CHUNK), jnp.int32),
      pltpu.VMEM((2 * GRP, CHUNK), jnp.int32),
  ]
  scratch += [pltpu.VMEM((CHUNK, H), jnp.float32) for _ in range(GRP)]
  if with_deg:
    out_type.append(jax.ShapeDtypeStruct((NC * DEG_W,), jnp.float32))
    scratch += [
        pltpu.VMEM((CHUNK,), jnp.float32),
        pltpu.VMEM((DEG_W // NS,), jnp.float32),
    ]
  scratch.append(pltpu.VMEM_SHARED((N_PAD, H), jnp.float32))
  if with_deg:
    scratch.append(pltpu.VMEM_SHARED((DEG_W,), jnp.float32))
  scratch += [pltpu.SemaphoreType.DMA for _ in range(2 * GRP)]
  if with_deg:
    scratch.append(pltpu.SemaphoreType.DMA)
  return pl.kernel(
      functools.partial(_sc_edge_body, with_deg),
      out_type=out_type, mesh=mesh, scratch_types=scratch,
      compiler_params=pltpu.CompilerParams(needs_layout_passes=False,
                                          use_tc_tiling_on_sc=False),
      name="sage_edge_agg_deg" if with_deg else "sage_edge_agg")


_sc_agg_deg = _make_sc_kernel(True)
_sc_agg = _make_sc_kernel(False)


BN = 1024


def _inv_deg(deg_ref):
  d = deg_ref[...]
  return 1.0 / jnp.maximum(d[0] + d[1], 1.0)


def _combine(acc_ref, deg_ref, y_ref):
  a = acc_ref[...]
  agg = a[0] + a[1]
  return jnp.maximum(agg * _inv_deg(deg_ref) + y_ref[...][:, H:], 0.0)


def _tc_pre_body(x_ref, w_ref, b_ref, y_ref):
  y_ref[...] = jnp.dot(x_ref[...], w_ref[...], **_DOT) + b_ref[...]


def _tc_mid_body(acc_ref, deg_ref, y_ref, w_ref, b_ref, yout_ref):
  h = _combine(acc_ref, deg_ref, y_ref)
  yout_ref[...] = jnp.dot(h, w_ref[...], **_DOT) + b_ref[...]


def _tc_fin_body(acc_ref, deg_ref, y_ref, lng_ref, lnb_ref, batch_ref,
                 u_ref, wpu_ref, wph_ref, bp_ref, wo_ref, bo_ref, out_ref):
  h = _combine(acc_ref, deg_ref, y_ref)
  mu = jnp.mean(h, axis=-1, keepdims=True)
  hc = h - mu
  var = jnp.mean(hc * hc, axis=-1, keepdims=True)
  hn = hc * lax.rsqrt(var + 1e-5) * lng_ref[...] + lnb_ref[...]
  gid = lax.broadcasted_iota(jnp.int32, (BN, G), 1)
  oneh = (batch_ref[...] == gid).astype(jnp.float32)
  up = jnp.dot(u_ref[...], wpu_ref[...], **_DOT)
  z = jnp.maximum(
      jnp.dot(hn, wph_ref[...], **_DOT) + jnp.dot(oneh, up, **_DOT)
      + bp_ref[...], 0.0)
  out_ref[...] = jnp.dot(z, wo_ref[...], **_DOT) + bo_ref[...]


def _row_spec(width):
  return pl.BlockSpec((BN, width), lambda i: (i, 0))


def _full_spec(shape):
  return pl.BlockSpec(shape, lambda i: tuple(0 for _ in shape))


def _acc_specs():
  return [pl.BlockSpec((2, BN, H), lambda i: (0, i, 0)),
          pl.BlockSpec((2, BN, 1), lambda i: (0, i, 0))]


_GRID = (10,)

_tc_pre = pl.pallas_call(
    _tc_pre_body,
    grid=_GRID,
    in_specs=[_row_spec(DIN), _full_spec((DIN, 2 * H)), _full_spec((1, 2 * H))],
    out_specs=_row_spec(2 * H),
    out_shape=jax.ShapeDtypeStruct((N, 2 * H), jnp.float32),
)

_tc_mid = pl.pallas_call(
    _tc_mid_body,
    grid=_GRID,
    in_specs=_acc_specs() + [_row_spec(2 * H), _full_spec((H, 2 * H)),
                             _full_spec((1, 2 * H))],
    out_specs=_row_spec(2 * H),
    out_shape=jax.ShapeDtypeStruct((N, 2 * H), jnp.float32),
)

_tc_fin = pl.pallas_call(
    _tc_fin_body,
    grid=_GRID,
    in_specs=_acc_specs() + [
        _row_spec(2 * H), _full_spec((1, H)), _full_spec((1, H)),
        _row_spec(1),
        _full_spec((G, 8)), _full_spec((8, H)),
        _full_spec((H, H)), _full_spec((1, H)),
        _full_spec((H, 2)), _full_spec((1, 2)),
    ],
    out_specs=_row_spec(2),
    out_shape=jax.ShapeDtypeStruct((N, 2), jnp.float32),
)


def kernel(x, edge_index, batch, u, Wl0, bl0, Wr0, Wl1, bl1, Wr1,
           Wl2, bl2, Wr2, ln_g, ln_b, Wp, bp, Wo, bo):
  pad = E_PAD - E
  src = jnp.concatenate([edge_index[0] * 2, jnp.zeros((pad,), jnp.int32)])
  dst = jnp.concatenate([edge_index[1], jnp.full((pad,), N, jnp.int32)])
  src2 = src.reshape(E_PAD // CHUNK, CHUNK)
  dst2 = dst.reshape(E_PAD // CHUNK, CHUNK)

  zH = jnp.zeros((H,), jnp.float32)
  w0 = jnp.concatenate([Wl0.T, Wr0.T], axis=1)
  w1 = jnp.concatenate([Wl1.T, Wr1.T], axis=1)
  w2 = jnp.concatenate([Wl2.T, Wr2.T], axis=1)
  b0 = jnp.concatenate([zH, bl0]).reshape(1, 2 * H)
  b1 = jnp.concatenate([zH, bl1]).reshape(1, 2 * H)
  b2 = jnp.concatenate([zH, bl2]).reshape(1, 2 * H)
  u_p = jnp.pad(u, ((0, 0), (0, 8 - GC)))
  wpu = jnp.pad(Wp[:, H:].T, ((0, 8 - GC), (0, 0)))
  wph = Wp[:, :H].T
  row = lambda v: v.reshape(1, -1)
  batch2 = batch.reshape(N, 1)

  y0 = _tc_pre(x, w0, b0)
  acc0, deg = _sc_agg_deg(y0.reshape(2 * N, H), src2, dst2)
  acc0 = acc0.reshape(NC, N_PAD, H)
  deg = deg.reshape(NC, DEG_W, 1)

  y1 = _tc_mid(acc0, deg, y0, w1, b1)
  acc1 = _sc_agg(y1.reshape(2 * N, H), src2, dst2)[0].reshape(NC, N_PAD, H)
  y2 = _tc_mid(acc1, deg, y1, w2, b2)
  acc2 = _sc_agg(y2.reshape(2 * N, H), src2, dst2)[0].reshape(NC, N_PAD, H)

  return _tc_fin(acc2, deg, y2, row(ln_g), row(ln_b), batch2,
                 u_p, wpu, wph, row(bp), Wo.T, row(bo))


# speedup vs baseline: 1.3419x; 1.1272x over previous
import functools

import jax
import jax.numpy as jnp
from jax import lax
from jax.experimental import pallas as pl
from jax.experimental.pallas import tpu as pltpu
from jax.experimental.pallas import tpu_sc as plsc

N = 10000
E = 320000
DIN = 128
H = 64
G = 16
GC = 3

NC = 2
NS = 16
CHUNK = 128
GRP = 4
CPT0 = 128
CPT1 = 32
E_PAD = NS * (CPT0 + CPT1) * CHUNK
RPT = 632
N_PAD = NS * RPT
DEG_W = 16384

_DOT = dict(precision=lax.Precision.HIGHEST, preferred_element_type=jnp.float32)


def _sc_edge_body(with_deg, *refs):
  if with_deg:
    (y_hbm, src_hbm, dst_hbm, acc_out, deg_out, src_v, dst_v,
     r0, r1, r2, r3, onesw_v, zw_v, acc_sh, deg_sh,
     g0, g1, g2, g3, s0, s1, s2, s3, dsem) = refs
  else:
    (y_hbm, src_hbm, dst_hbm, acc_out, src_v, dst_v,
     r0, r1, r2, r3, acc_sh,
     g0, g1, g2, g3, s0, s1, s2, s3) = refs
  rows = [r0, r1, r2, r3]
  gsem = [g0, g1, g2, g3]
  ssem = [s0, s1, s2, s3]
  c = lax.axis_index("c")
  s = lax.axis_index("s")
  cbase = jnp.where(c == 0, s * CPT0, NS * CPT0 + s * CPT1)
  ngrp = jnp.where(c == 0, CPT0 // (2 * GRP), CPT1 // (2 * GRP))
  zeros16 = jnp.zeros((16,), jnp.float32)
  ones16 = jnp.ones((16,), jnp.float32)

  def _z(i, _):
    r0[i >> 2, pl.ds((i & 3) * 16, 16)] = zeros16
    return 0
  lax.fori_loop(0, CHUNK * 4, _z, 0)
  for k in range(RPT // CHUNK):
    pltpu.sync_copy(r0, acc_sh.at[pl.ds(s * RPT + k * CHUNK, CHUNK)])
  rem = RPT - (RPT // CHUNK) * CHUNK
  pltpu.sync_copy(r0.at[pl.ds(0, rem)],
                  acc_sh.at[pl.ds(s * RPT + RPT - rem, rem)])
  if with_deg:
    for g in range(CHUNK // 16):
      onesw_v[pl.ds(g * 16, 16)] = ones16
    def _zd(i, _):
      zw_v[pl.ds(i * 16, 16)] = zeros16
      return 0
    lax.fori_loop(0, DEG_W // NS // 16, _zd, 0)
    pltpu.sync_copy(zw_v, deg_sh.at[pl.ds(s * (DEG_W // NS), DEG_W // NS)])
  plsc.subcore_barrier()

  def _super(u, _):
    pltpu.sync_copy(src_hbm.at[pl.ds(cbase + u * 2 * GRP, 2 * GRP)], src_v)
    pltpu.sync_copy(dst_hbm.at[pl.ds(cbase + u * 2 * GRP, 2 * GRP)], dst_v)
    for half in (0, GRP):
      gd = [pltpu.async_copy(y_hbm.at[src_v.at[half + b]], rows[b], gsem[b])
            for b in range(GRP)]
      sd = []
      dd = []
      for b in range(GRP):
        gd[b].wait()
        sd.append(pltpu.async_copy(rows[b], acc_sh.at[dst_v.at[half + b]],
                                   ssem[b], add=True))
        if with_deg:
          dd.append(pltpu.async_copy(onesw_v, deg_sh.at[dst_v.at[half + b]],
                                     dsem, add=True))
      for d in sd + dd:
        d.wait()
    return 0
  lax.fori_loop(0, ngrp, _super, 0)
  plsc.subcore_barrier()

  off = c * N_PAD + s * RPT
  pltpu.sync_copy(acc_sh.at[pl.ds(s * RPT, RPT)], acc_out.at[pl.ds(off, RPT)])
  if with_deg:
    w = DEG_W // NS
    pltpu.sync_copy(deg_sh.at[pl.ds(s * w, w)],
                    deg_out.at[pl.ds(c * DEG_W + s * w, w)])


def _make_sc_kernel(with_deg):
  mesh = plsc.VectorSubcoreMesh(core_axis_name="c", subcore_axis_name="s")
  out_type = [jax.ShapeDtypeStruct((NC * N_PAD, H), jnp.float32)]
  scratch = [
      pltpu.VMEM((2 * GRP, CHUNK), jnp.int32),
      pltpu.VMEM((2 * GRP, CHUNK), jnp.int32),
  ]
  scratch += [pltpu.VMEM((CHUNK, H), jnp.float32) for _ in range(GRP)]
  if with_deg:
    out_type.append(jax.ShapeDtypeStruct((NC * DEG_W,), jnp.float32))
    scratch += [
        pltpu.VMEM((CHUNK,), jnp.float32),
        pltpu.VMEM((DEG_W // NS,), jnp.float32),
    ]
  scratch.append(pltpu.VMEM_SHARED((N_PAD, H), jnp.float32))
  if with_deg:
    scratch.append(pltpu.VMEM_SHARED((DEG_W,), jnp.float32))
  scratch += [pltpu.SemaphoreType.DMA for _ in range(2 * GRP)]
  if with_deg:
    scratch.append(pltpu.SemaphoreType.DMA)
  return pl.kernel(
      functools.partial(_sc_edge_body, with_deg),
      out_type=out_type, mesh=mesh, scratch_types=scratch,
      compiler_params=pltpu.CompilerParams(needs_layout_passes=False,
                                          use_tc_tiling_on_sc=False),
      name="sage_edge_agg_deg" if with_deg else "sage_edge_agg")


_sc_agg_deg = _make_sc_kernel(True)
_sc_agg = _make_sc_kernel(False)


BN = 1024


def _inv_deg(deg_ref):
  d = deg_ref[...]
  return 1.0 / jnp.maximum(d[0] + d[1], 1.0)


def _combine(acc_ref, deg_ref, y_ref):
  a = acc_ref[...]
  agg = a[0] + a[1]
  return jnp.maximum(agg * _inv_deg(deg_ref) + y_ref[...][:, H:], 0.0)


def _tc_pre_body(x_ref, w_ref, b_ref, y_ref):
  y_ref[...] = jnp.dot(x_ref[...], w_ref[...], **_DOT) + b_ref[...]


def _tc_mid_body(acc_ref, deg_ref, y_ref, w_ref, b_ref, yout_ref):
  h = _combine(acc_ref, deg_ref, y_ref)
  yout_ref[...] = jnp.dot(h, w_ref[...], **_DOT) + b_ref[...]


def _tc_fin_body(acc_ref, deg_ref, y_ref, lng_ref, lnb_ref, batch_ref,
                 u_ref, wpu_ref, wph_ref, bp_ref, wo_ref, bo_ref, out_ref):
  h = _combine(acc_ref, deg_ref, y_ref)
  mu = jnp.mean(h, axis=-1, keepdims=True)
  hc = h - mu
  var = jnp.mean(hc * hc, axis=-1, keepdims=True)
  hn = hc * lax.rsqrt(var + 1e-5) * lng_ref[...] + lnb_ref[...]
  gid = lax.broadcasted_iota(jnp.int32, (BN, G), 1)
  oneh = (batch_ref[...] == gid).astype(jnp.float32)
  up = jnp.dot(u_ref[...], wpu_ref[...], **_DOT)
  z = jnp.maximum(
      jnp.dot(hn, wph_ref[...], **_DOT) + jnp.dot(oneh, up, **_DOT)
      + bp_ref[...], 0.0)
  out_ref[...] = jnp.dot(z, wo_ref[...], **_DOT) + bo_ref[...]


def _row_spec(width):
  return pl.BlockSpec((BN, width), lambda i: (i, 0))


def _full_spec(shape):
  return pl.BlockSpec(shape, lambda i: tuple(0 for _ in shape))


def _acc_specs():
  return [pl.BlockSpec((2, BN, H), lambda i: (0, i, 0)),
          pl.BlockSpec((2, BN, 1), lambda i: (0, i, 0))]


_GRID = (10,)

_tc_pre = pl.pallas_call(
    _tc_pre_body,
    grid=_GRID,
    in_specs=[_row_spec(DIN), _full_spec((DIN, 2 * H)), _full_spec((1, 2 * H))],
    out_specs=_row_spec(2 * H),
    out_shape=jax.ShapeDtypeStruct((N, 2 * H), jnp.float32),
)

_tc_mid = pl.pallas_call(
    _tc_mid_body,
    grid=_GRID,
    in_specs=_acc_specs() + [_row_spec(2 * H), _full_spec((H, 2 * H)),
                             _full_spec((1, 2 * H))],
    out_specs=_row_spec(2 * H),
    out_shape=jax.ShapeDtypeStruct((N, 2 * H), jnp.float32),
)

_tc_fin = pl.pallas_call(
    _tc_fin_body,
    grid=_GRID,
    in_specs=_acc_specs() + [
        _row_spec(2 * H), _full_spec((1, H)), _full_spec((1, H)),
        _row_spec(1),
        _full_spec((G, 8)), _full_spec((8, H)),
        _full_spec((H, H)), _full_spec((1, H)),
        _full_spec((H, 2)), _full_spec((1, 2)),
    ],
    out_specs=_row_spec(2),
    out_shape=jax.ShapeDtypeStruct((N, 2), jnp.float32),
)


def kernel(x, edge_index, batch, u, Wl0, bl0, Wr0, Wl1, bl1, Wr1,
           Wl2, bl2, Wr2, ln_g, ln_b, Wp, bp, Wo, bo):
  pad = E_PAD - E
  src = jnp.concatenate([edge_index[0] * 2, jnp.zeros((pad,), jnp.int32)])
  dst = jnp.concatenate([edge_index[1], jnp.full((pad,), N, jnp.int32)])
  src2 = src.reshape(E_PAD // CHUNK, CHUNK)
  dst2 = dst.reshape(E_PAD // CHUNK, CHUNK)

  zH = jnp.zeros((H,), jnp.float32)
  w0 = jnp.concatenate([Wl0.T, Wr0.T], axis=1)
  w1 = jnp.concatenate([Wl1.T, Wr1.T], axis=1)
  w2 = jnp.concatenate([Wl2.T, Wr2.T], axis=1)
  b0 = jnp.concatenate([zH, bl0]).reshape(1, 2 * H)
  b1 = jnp.concatenate([zH, bl1]).reshape(1, 2 * H)
  b2 = jnp.concatenate([zH, bl2]).reshape(1, 2 * H)
  u_p = jnp.pad(u, ((0, 0), (0, 8 - GC)))
  wpu = jnp.pad(Wp[:, H:].T, ((0, 8 - GC), (0, 0)))
  wph = Wp[:, :H].T
  row = lambda v: v.reshape(1, -1)
  batch2 = batch.reshape(N, 1)

  y0 = _tc_pre(x, w0, b0)
  acc0, deg = _sc_agg_deg(y0.reshape(2 * N, H), src2, dst2)
  acc0 = acc0.reshape(NC, N_PAD, H)
  deg = deg.reshape(NC, DEG_W, 1)

  y1 = _tc_mid(acc0, deg, y0, w1, b1)
  acc1 = _sc_agg(y1.reshape(2 * N, H), src2, dst2)[0].reshape(NC, N_PAD, H)
  y2 = _tc_mid(acc1, deg, y1, w2, b2)
  acc2 = _sc_agg(y2.reshape(2 * N, H), src2, dst2)[0].reshape(NC, N_PAD, H)

  return _tc_fin(acc2, deg, y2, row(ln_g), row(ln_b), batch2,
                 u_p, wpu, wph, row(bp), Wo.T, row(bo))


# speedup vs baseline: 1.3605x; 1.0139x over previous
import functools

import jax
import jax.numpy as jnp
from jax import lax
from jax.experimental import pallas as pl
from jax.experimental.pallas import tpu as pltpu
from jax.experimental.pallas import tpu_sc as plsc

N = 10000
E = 320000
DIN = 128
H = 64
G = 16
GC = 3

NC = 2
NS = 16
CHUNK = 128
GRP = 4
CPT0 = 136
CPT1 = 24
E_PAD = NS * (CPT0 + CPT1) * CHUNK
RPT = 632
N_PAD = NS * RPT
DEG_W = 16384

_DOT = dict(precision=lax.Precision.HIGHEST, preferred_element_type=jnp.float32)


def _sc_edge_body(with_deg, *refs):
  if with_deg:
    (y_hbm, src_hbm, dst_hbm, acc_out, deg_out, src_v, dst_v,
     r0, r1, r2, r3, onesw_v, zw_v, acc_sh, deg_sh,
     g0, g1, g2, g3, s0, s1, s2, s3, dsem) = refs
  else:
    (y_hbm, src_hbm, dst_hbm, acc_out, src_v, dst_v,
     r0, r1, r2, r3, acc_sh,
     g0, g1, g2, g3, s0, s1, s2, s3) = refs
  rows = [r0, r1, r2, r3]
  gsem = [g0, g1, g2, g3]
  ssem = [s0, s1, s2, s3]
  c = lax.axis_index("c")
  s = lax.axis_index("s")
  cbase = jnp.where(c == 0, s * CPT0, NS * CPT0 + s * CPT1)
  ngrp = jnp.where(c == 0, CPT0 // (2 * GRP), CPT1 // (2 * GRP))
  zeros16 = jnp.zeros((16,), jnp.float32)
  ones16 = jnp.ones((16,), jnp.float32)

  def _z(i, _):
    r0[i >> 2, pl.ds((i & 3) * 16, 16)] = zeros16
    return 0
  lax.fori_loop(0, CHUNK * 4, _z, 0)
  for k in range(RPT // CHUNK):
    pltpu.sync_copy(r0, acc_sh.at[pl.ds(s * RPT + k * CHUNK, CHUNK)])
  rem = RPT - (RPT // CHUNK) * CHUNK
  pltpu.sync_copy(r0.at[pl.ds(0, rem)],
                  acc_sh.at[pl.ds(s * RPT + RPT - rem, rem)])
  if with_deg:
    for g in range(CHUNK // 16):
      onesw_v[pl.ds(g * 16, 16)] = ones16
    def _zd(i, _):
      zw_v[pl.ds(i * 16, 16)] = zeros16
      return 0
    lax.fori_loop(0, DEG_W // NS // 16, _zd, 0)
    pltpu.sync_copy(zw_v, deg_sh.at[pl.ds(s * (DEG_W // NS), DEG_W // NS)])
  plsc.subcore_barrier()

  def _super(u, _):
    pltpu.sync_copy(src_hbm.at[pl.ds(cbase + u * 2 * GRP, 2 * GRP)], src_v)
    pltpu.sync_copy(dst_hbm.at[pl.ds(cbase + u * 2 * GRP, 2 * GRP)], dst_v)
    for half in (0, GRP):
      gd = [pltpu.async_copy(y_hbm.at[src_v.at[half + b]], rows[b], gsem[b])
            for b in range(GRP)]
      sd = []
      dd = []
      for b in range(GRP):
        gd[b].wait()
        sd.append(pltpu.async_copy(rows[b], acc_sh.at[dst_v.at[half + b]],
                                   ssem[b], add=True))
        if with_deg:
          dd.append(pltpu.async_copy(onesw_v, deg_sh.at[dst_v.at[half + b]],
                                     dsem, add=True))
      for d in sd + dd:
        d.wait()
    return 0
  lax.fori_loop(0, ngrp, _super, 0)
  plsc.subcore_barrier()

  off = c * N_PAD + s * RPT
  pltpu.sync_copy(acc_sh.at[pl.ds(s * RPT, RPT)], acc_out.at[pl.ds(off, RPT)])
  if with_deg:
    w = DEG_W // NS
    pltpu.sync_copy(deg_sh.at[pl.ds(s * w, w)],
                    deg_out.at[pl.ds(c * DEG_W + s * w, w)])


def _make_sc_kernel(with_deg):
  mesh = plsc.VectorSubcoreMesh(core_axis_name="c", subcore_axis_name="s")
  out_type = [jax.ShapeDtypeStruct((NC * N_PAD, H), jnp.float32)]
  scratch = [
      pltpu.VMEM((2 * GRP, CHUNK), jnp.int32),
      pltpu.VMEM((2 * GRP, CHUNK), jnp.int32),
  ]
  scratch += [pltpu.VMEM((CHUNK, H), jnp.float32) for _ in range(GRP)]
  if with_deg:
    out_type.append(jax.ShapeDtypeStruct((NC * DEG_W,), jnp.float32))
    scratch += [
        pltpu.VMEM((CHUNK,), jnp.float32),
        pltpu.VMEM((DEG_W // NS,), jnp.float32),
    ]
  scratch.append(pltpu.VMEM_SHARED((N_PAD, H), jnp.float32))
  if with_deg:
    scratch.append(pltpu.VMEM_SHARED((DEG_W,), jnp.float32))
  scratch += [pltpu.SemaphoreType.DMA for _ in range(2 * GRP)]
  if with_deg:
    scratch.append(pltpu.SemaphoreType.DMA)
  return pl.kernel(
      functools.partial(_sc_edge_body, with_deg),
      out_type=out_type, mesh=mesh, scratch_types=scratch,
      compiler_params=pltpu.CompilerParams(needs_layout_passes=False,
                                          use_tc_tiling_on_sc=False),
      name="sage_edge_agg_deg" if with_deg else "sage_edge_agg")


_sc_agg_deg = _make_sc_kernel(True)
_sc_agg = _make_sc_kernel(False)


BN = 1024


def _inv_deg(deg_ref):
  d = deg_ref[...]
  return 1.0 / jnp.maximum(d[0] + d[1], 1.0)


def _combine(acc_ref, deg_ref, y_ref):
  a = acc_ref[...]
  agg = a[0] + a[1]
  return jnp.maximum(agg * _inv_deg(deg_ref) + y_ref[...][:, H:], 0.0)


def _tc_pre_body(x_ref, w_ref, b_ref, y_ref):
  y_ref[...] = jnp.dot(x_ref[...], w_ref[...], **_DOT) + b_ref[...]


def _tc_mid_body(acc_ref, deg_ref, y_ref, w_ref, b_ref, yout_ref):
  h = _combine(acc_ref, deg_ref, y_ref)
  yout_ref[...] = jnp.dot(h, w_ref[...], **_DOT) + b_ref[...]


def _tc_fin_body(acc_ref, deg_ref, y_ref, lng_ref, lnb_ref, batch_ref,
                 u_ref, wpu_ref, wph_ref, bp_ref, wo_ref, bo_ref, out_ref):
  h = _combine(acc_ref, deg_ref, y_ref)
  mu = jnp.mean(h, axis=-1, keepdims=True)
  hc = h - mu
  var = jnp.mean(hc * hc, axis=-1, keepdims=True)
  hn = hc * lax.rsqrt(var + 1e-5) * lng_ref[...] + lnb_ref[...]
  gid = lax.broadcasted_iota(jnp.int32, (BN, G), 1)
  oneh = (batch_ref[...] == gid).astype(jnp.float32)
  up = jnp.dot(u_ref[...], wpu_ref[...], **_DOT)
  z = jnp.maximum(
      jnp.dot(hn, wph_ref[...], **_DOT) + jnp.dot(oneh, up, **_DOT)
      + bp_ref[...], 0.0)
  out_ref[...] = jnp.dot(z, wo_ref[...], **_DOT) + bo_ref[...]


def _row_spec(width):
  return pl.BlockSpec((BN, width), lambda i: (i, 0))


def _full_spec(shape):
  return pl.BlockSpec(shape, lambda i: tuple(0 for _ in shape))


def _acc_specs():
  return [pl.BlockSpec((2, BN, H), lambda i: (0, i, 0)),
          pl.BlockSpec((2, BN, 1), lambda i: (0, i, 0))]


_GRID = (10,)

_tc_pre = pl.pallas_call(
    _tc_pre_body,
    grid=_GRID,
    in_specs=[_row_spec(DIN), _full_spec((DIN, 2 * H)), _full_spec((1, 2 * H))],
    out_specs=_row_spec(2 * H),
    out_shape=jax.ShapeDtypeStruct((N, 2 * H), jnp.float32),
)

_tc_mid = pl.pallas_call(
    _tc_mid_body,
    grid=_GRID,
    in_specs=_acc_specs() + [_row_spec(2 * H), _full_spec((H, 2 * H)),
                             _full_spec((1, 2 * H))],
    out_specs=_row_spec(2 * H),
    out_shape=jax.ShapeDtypeStruct((N, 2 * H), jnp.float32),
)

_tc_fin = pl.pallas_call(
    _tc_fin_body,
    grid=_GRID,
    in_specs=_acc_specs() + [
        _row_spec(2 * H), _full_spec((1, H)), _full_spec((1, H)),
        _row_spec(1),
        _full_spec((G, 8)), _full_spec((8, H)),
        _full_spec((H, H)), _full_spec((1, H)),
        _full_spec((H, 2)), _full_spec((1, 2)),
    ],
    out_specs=_row_spec(2),
    out_shape=jax.ShapeDtypeStruct((N, 2), jnp.float32),
)


def kernel(x, edge_index, batch, u, Wl0, bl0, Wr0, Wl1, bl1, Wr1,
           Wl2, bl2, Wr2, ln_g, ln_b, Wp, bp, Wo, bo):
  pad = E_PAD - E
  src = jnp.concatenate([edge_index[0] * 2, jnp.zeros((pad,), jnp.int32)])
  dst = jnp.concatenate([edge_index[1], jnp.full((pad,), N, jnp.int32)])
  src2 = src.reshape(E_PAD // CHUNK, CHUNK)
  dst2 = dst.reshape(E_PAD // CHUNK, CHUNK)

  zH = jnp.zeros((H,), jnp.float32)
  w0 = jnp.concatenate([Wl0.T, Wr0.T], axis=1)
  w1 = jnp.concatenate([Wl1.T, Wr1.T], axis=1)
  w2 = jnp.concatenate([Wl2.T, Wr2.T], axis=1)
  b0 = jnp.concatenate([zH, bl0]).reshape(1, 2 * H)
  b1 = jnp.concatenate([zH, bl1]).reshape(1, 2 * H)
  b2 = jnp.concatenate([zH, bl2]).reshape(1, 2 * H)
  u_p = jnp.pad(u, ((0, 0), (0, 8 - GC)))
  wpu = jnp.pad(Wp[:, H:].T, ((0, 8 - GC), (0, 0)))
  wph = Wp[:, :H].T
  row = lambda v: v.reshape(1, -1)
  batch2 = batch.reshape(N, 1)

  y0 = _tc_pre(x, w0, b0)
  acc0, deg = _sc_agg_deg(y0.reshape(2 * N, H), src2, dst2)
  acc0 = acc0.reshape(NC, N_PAD, H)
  deg = deg.reshape(NC, DEG_W, 1)

  y1 = _tc_mid(acc0, deg, y0, w1, b1)
  acc1 = _sc_agg(y1.reshape(2 * N, H), src2, dst2)[0].reshape(NC, N_PAD, H)
  y2 = _tc_mid(acc1, deg, y1, w2, b2)
  acc2 = _sc_agg(y2.reshape(2 * N, H), src2, dst2)[0].reshape(NC, N_PAD, H)

  return _tc_fin(acc2, deg, y2, row(ln_g), row(ln_b), batch2,
                 u_p, wpu, wph, row(bp), Wo.T, row(bo))


# speedup vs baseline: 1.4167x; 1.0413x over previous
import functools

import jax
import jax.numpy as jnp
from jax import lax
from jax.experimental import pallas as pl
from jax.experimental.pallas import tpu as pltpu
from jax.experimental.pallas import tpu_sc as plsc

N = 10000
E = 320000
DIN = 128
H = 64
G = 16
GC = 3

NC = 2
NS = 16
CHUNK = 128
GRP = 4
CPT0 = 144
CPT1 = 16
E_PAD = NS * (CPT0 + CPT1) * CHUNK
RPT = 632
N_PAD = NS * RPT
DEG_W = 16384

_DOT = dict(precision=lax.Precision.HIGHEST, preferred_element_type=jnp.float32)


def _sc_edge_body(with_deg, *refs):
  if with_deg:
    (y_hbm, src_hbm, dst_hbm, acc_out, deg_out, src_v, dst_v,
     r0, r1, r2, r3, onesw_v, zw_v, acc_sh, deg_sh,
     g0, g1, g2, g3, s0, s1, s2, s3, dsem) = refs
  else:
    (y_hbm, src_hbm, dst_hbm, acc_out, src_v, dst_v,
     r0, r1, r2, r3, acc_sh,
     g0, g1, g2, g3, s0, s1, s2, s3) = refs
  rows = [r0, r1, r2, r3]
  gsem = [g0, g1, g2, g3]
  ssem = [s0, s1, s2, s3]
  c = lax.axis_index("c")
  s = lax.axis_index("s")
  cbase = jnp.where(c == 0, s * CPT0, NS * CPT0 + s * CPT1)
  ngrp = jnp.where(c == 0, CPT0 // (2 * GRP), CPT1 // (2 * GRP))
  zeros16 = jnp.zeros((16,), jnp.float32)
  ones16 = jnp.ones((16,), jnp.float32)

  def _z(i, _):
    r0[i >> 2, pl.ds((i & 3) * 16, 16)] = zeros16
    return 0
  lax.fori_loop(0, CHUNK * 4, _z, 0)
  for k in range(RPT // CHUNK):
    pltpu.sync_copy(r0, acc_sh.at[pl.ds(s * RPT + k * CHUNK, CHUNK)])
  rem = RPT - (RPT // CHUNK) * CHUNK
  pltpu.sync_copy(r0.at[pl.ds(0, rem)],
                  acc_sh.at[pl.ds(s * RPT + RPT - rem, rem)])
  if with_deg:
    for g in range(CHUNK // 16):
      onesw_v[pl.ds(g * 16, 16)] = ones16
    def _zd(i, _):
      zw_v[pl.ds(i * 16, 16)] = zeros16
      return 0
    lax.fori_loop(0, DEG_W // NS // 16, _zd, 0)
    pltpu.sync_copy(zw_v, deg_sh.at[pl.ds(s * (DEG_W // NS), DEG_W // NS)])
  plsc.subcore_barrier()

  def _super(u, _):
    pltpu.sync_copy(src_hbm.at[pl.ds(cbase + u * 2 * GRP, 2 * GRP)], src_v)
    pltpu.sync_copy(dst_hbm.at[pl.ds(cbase + u * 2 * GRP, 2 * GRP)], dst_v)
    for half in (0, GRP):
      gd = [pltpu.async_copy(y_hbm.at[src_v.at[half + b]], rows[b], gsem[b])
            for b in range(GRP)]
      sd = []
      dd = []
      for b in range(GRP):
        gd[b].wait()
        sd.append(pltpu.async_copy(rows[b], acc_sh.at[dst_v.at[half + b]],
                                   ssem[b], add=True))
        if with_deg:
          dd.append(pltpu.async_copy(onesw_v, deg_sh.at[dst_v.at[half + b]],
                                     dsem, add=True))
      for d in sd + dd:
        d.wait()
    return 0
  lax.fori_loop(0, ngrp, _super, 0)
  plsc.subcore_barrier()

  off = c * N_PAD + s * RPT
  pltpu.sync_copy(acc_sh.at[pl.ds(s * RPT, RPT)], acc_out.at[pl.ds(off, RPT)])
  if with_deg:
    w = DEG_W // NS
    pltpu.sync_copy(deg_sh.at[pl.ds(s * w, w)],
                    deg_out.at[pl.ds(c * DEG_W + s * w, w)])


def _make_sc_kernel(with_deg):
  mesh = plsc.VectorSubcoreMesh(core_axis_name="c", subcore_axis_name="s")
  out_type = [jax.ShapeDtypeStruct((NC * N_PAD, H), jnp.float32)]
  scratch = [
      pltpu.VMEM((2 * GRP, CHUNK), jnp.int32),
      pltpu.VMEM((2 * GRP, CHUNK), jnp.int32),
  ]
  scratch += [pltpu.VMEM((CHUNK, H), jnp.float32) for _ in range(GRP)]
  if with_deg:
    out_type.append(jax.ShapeDtypeStruct((NC * DEG_W,), jnp.float32))
    scratch += [
        pltpu.VMEM((CHUNK,), jnp.float32),
        pltpu.VMEM((DEG_W // NS,), jnp.float32),
    ]
  scratch.append(pltpu.VMEM_SHARED((N_PAD, H), jnp.float32))
  if with_deg:
    scratch.append(pltpu.VMEM_SHARED((DEG_W,), jnp.float32))
  scratch += [pltpu.SemaphoreType.DMA for _ in range(2 * GRP)]
  if with_deg:
    scratch.append(pltpu.SemaphoreType.DMA)
  return pl.kernel(
      functools.partial(_sc_edge_body, with_deg),
      out_type=out_type, mesh=mesh, scratch_types=scratch,
      compiler_params=pltpu.CompilerParams(needs_layout_passes=False,
                                          use_tc_tiling_on_sc=False),
      name="sage_edge_agg_deg" if with_deg else "sage_edge_agg")


_sc_agg_deg = _make_sc_kernel(True)
_sc_agg = _make_sc_kernel(False)


BN = 1024


def _inv_deg(deg_ref):
  d = deg_ref[...]
  return 1.0 / jnp.maximum(d[0] + d[1], 1.0)


def _combine(acc_ref, deg_ref, y_ref):
  a = acc_ref[...]
  agg = a[0] + a[1]
  return jnp.maximum(agg * _inv_deg(deg_ref) + y_ref[...][:, H:], 0.0)


def _tc_pre_body(x_ref, w_ref, b_ref, y_ref):
  y_ref[...] = jnp.dot(x_ref[...], w_ref[...], **_DOT) + b_ref[...]


def _tc_mid_body(acc_ref, deg_ref, y_ref, w_ref, b_ref, yout_ref):
  h = _combine(acc_ref, deg_ref, y_ref)
  yout_ref[...] = jnp.dot(h, w_ref[...], **_DOT) + b_ref[...]


def _tc_fin_body(acc_ref, deg_ref, y_ref, lng_ref, lnb_ref, batch_ref,
                 u_ref, wpu_ref, wph_ref, bp_ref, wo_ref, bo_ref, out_ref):
  h = _combine(acc_ref, deg_ref, y_ref)
  mu = jnp.mean(h, axis=-1, keepdims=True)
  hc = h - mu
  var = jnp.mean(hc * hc, axis=-1, keepdims=True)
  hn = hc * lax.rsqrt(var + 1e-5) * lng_ref[...] + lnb_ref[...]
  gid = lax.broadcasted_iota(jnp.int32, (BN, G), 1)
  oneh = (batch_ref[...] == gid).astype(jnp.float32)
  up = jnp.dot(u_ref[...], wpu_ref[...], **_DOT)
  z = jnp.maximum(
      jnp.dot(hn, wph_ref[...], **_DOT) + jnp.dot(oneh, up, **_DOT)
      + bp_ref[...], 0.0)
  out_ref[...] = jnp.dot(z, wo_ref[...], **_DOT) + bo_ref[...]


def _row_spec(width):
  return pl.BlockSpec((BN, width), lambda i: (i, 0))


def _full_spec(shape):
  return pl.BlockSpec(shape, lambda i: tuple(0 for _ in shape))


def _acc_specs():
  return [pl.BlockSpec((2, BN, H), lambda i: (0, i, 0)),
          pl.BlockSpec((2, BN, 1), lambda i: (0, i, 0))]


_GRID = (10,)

_tc_pre = pl.pallas_call(
    _tc_pre_body,
    grid=_GRID,
    in_specs=[_row_spec(DIN), _full_spec((DIN, 2 * H)), _full_spec((1, 2 * H))],
    out_specs=_row_spec(2 * H),
    out_shape=jax.ShapeDtypeStruct((N, 2 * H), jnp.float32),
)

_tc_mid = pl.pallas_call(
    _tc_mid_body,
    grid=_GRID,
    in_specs=_acc_specs() + [_row_spec(2 * H), _full_spec((H, 2 * H)),
                             _full_spec((1, 2 * H))],
    out_specs=_row_spec(2 * H),
    out_shape=jax.ShapeDtypeStruct((N, 2 * H), jnp.float32),
)

_tc_fin = pl.pallas_call(
    _tc_fin_body,
    grid=_GRID,
    in_specs=_acc_specs() + [
        _row_spec(2 * H), _full_spec((1, H)), _full_spec((1, H)),
        _row_spec(1),
        _full_spec((G, 8)), _full_spec((8, H)),
        _full_spec((H, H)), _full_spec((1, H)),
        _full_spec((H, 2)), _full_spec((1, 2)),
    ],
    out_specs=_row_spec(2),
    out_shape=jax.ShapeDtypeStruct((N, 2), jnp.float32),
)


def kernel(x, edge_index, batch, u, Wl0, bl0, Wr0, Wl1, bl1, Wr1,
           Wl2, bl2, Wr2, ln_g, ln_b, Wp, bp, Wo, bo):
  pad = E_PAD - E
  src = jnp.concatenate([edge_index[0] * 2, jnp.zeros((pad,), jnp.int32)])
  dst = jnp.concatenate([edge_index[1], jnp.full((pad,), N, jnp.int32)])
  src2 = src.reshape(E_PAD // CHUNK, CHUNK)
  dst2 = dst.reshape(E_PAD // CHUNK, CHUNK)

  zH = jnp.zeros((H,), jnp.float32)
  w0 = jnp.concatenate([Wl0.T, Wr0.T], axis=1)
  w1 = jnp.concatenate([Wl1.T, Wr1.T], axis=1)
  w2 = jnp.concatenate([Wl2.T, Wr2.T], axis=1)
  b0 = jnp.concatenate([zH, bl0]).reshape(1, 2 * H)
  b1 = jnp.concatenate([zH, bl1]).reshape(1, 2 * H)
  b2 = jnp.concatenate([zH, bl2]).reshape(1, 2 * H)
  u_p = jnp.pad(u, ((0, 0), (0, 8 - GC)))
  wpu = jnp.pad(Wp[:, H:].T, ((0, 8 - GC), (0, 0)))
  wph = Wp[:, :H].T
  row = lambda v: v.reshape(1, -1)
  batch2 = batch.reshape(N, 1)

  y0 = _tc_pre(x, w0, b0)
  acc0, deg = _sc_agg_deg(y0.reshape(2 * N, H), src2, dst2)
  acc0 = acc0.reshape(NC, N_PAD, H)
  deg = deg.reshape(NC, DEG_W, 1)

  y1 = _tc_mid(acc0, deg, y0, w1, b1)
  acc1 = _sc_agg(y1.reshape(2 * N, H), src2, dst2)[0].reshape(NC, N_PAD, H)
  y2 = _tc_mid(acc1, deg, y1, w2, b2)
  acc2 = _sc_agg(y2.reshape(2 * N, H), src2, dst2)[0].reshape(NC, N_PAD, H)

  return _tc_fin(acc2, deg, y2, row(ln_g), row(ln_b), batch2,
                 u_p, wpu, wph, row(bp), Wo.T, row(bo))


# speedup vs baseline: 1.4265x; 1.0069x over previous
import functools

import jax
import jax.numpy as jnp
from jax import lax
from jax.experimental import pallas as pl
from jax.experimental.pallas import tpu as pltpu
from jax.experimental.pallas import tpu_sc as plsc

N = 10000
E = 320000
DIN = 128
H = 64
G = 16
GC = 3

NC = 2
NS = 16
CHUNK = 128
GRP = 4
CPT0 = 152
CPT1 = 8
E_PAD = NS * (CPT0 + CPT1) * CHUNK
RPT = 632
N_PAD = NS * RPT
DEG_W = 16384

_DOT = dict(precision=lax.Precision.HIGHEST, preferred_element_type=jnp.float32)


def _sc_edge_body(with_deg, *refs):
  if with_deg:
    (y_hbm, src_hbm, dst_hbm, acc_out, deg_out, src_v, dst_v,
     r0, r1, r2, r3, onesw_v, zw_v, acc_sh, deg_sh,
     g0, g1, g2, g3, s0, s1, s2, s3, dsem) = refs
  else:
    (y_hbm, src_hbm, dst_hbm, acc_out, src_v, dst_v,
     r0, r1, r2, r3, acc_sh,
     g0, g1, g2, g3, s0, s1, s2, s3) = refs
  rows = [r0, r1, r2, r3]
  gsem = [g0, g1, g2, g3]
  ssem = [s0, s1, s2, s3]
  c = lax.axis_index("c")
  s = lax.axis_index("s")
  cbase = jnp.where(c == 0, s * CPT0, NS * CPT0 + s * CPT1)
  ngrp = jnp.where(c == 0, CPT0 // (2 * GRP), CPT1 // (2 * GRP))
  zeros16 = jnp.zeros((16,), jnp.float32)
  ones16 = jnp.ones((16,), jnp.float32)

  def _z(i, _):
    r0[i >> 2, pl.ds((i & 3) * 16, 16)] = zeros16
    return 0
  lax.fori_loop(0, CHUNK * 4, _z, 0)
  for k in range(RPT // CHUNK):
    pltpu.sync_copy(r0, acc_sh.at[pl.ds(s * RPT + k * CHUNK, CHUNK)])
  rem = RPT - (RPT // CHUNK) * CHUNK
  pltpu.sync_copy(r0.at[pl.ds(0, rem)],
                  acc_sh.at[pl.ds(s * RPT + RPT - rem, rem)])
  if with_deg:
    for g in range(CHUNK // 16):
      onesw_v[pl.ds(g * 16, 16)] = ones16
    def _zd(i, _):
      zw_v[pl.ds(i * 16, 16)] = zeros16
      return 0
    lax.fori_loop(0, DEG_W // NS // 16, _zd, 0)
    pltpu.sync_copy(zw_v, deg_sh.at[pl.ds(s * (DEG_W // NS), DEG_W // NS)])
  plsc.subcore_barrier()

  def _super(u, _):
    pltpu.sync_copy(src_hbm.at[pl.ds(cbase + u * 2 * GRP, 2 * GRP)], src_v)
    pltpu.sync_copy(dst_hbm.at[pl.ds(cbase + u * 2 * GRP, 2 * GRP)], dst_v)
    for half in (0, GRP):
      gd = [pltpu.async_copy(y_hbm.at[src_v.at[half + b]], rows[b], gsem[b])
            for b in range(GRP)]
      sd = []
      dd = []
      for b in range(GRP):
        gd[b].wait()
        sd.append(pltpu.async_copy(rows[b], acc_sh.at[dst_v.at[half + b]],
                                   ssem[b], add=True))
        if with_deg:
          dd.append(pltpu.async_copy(onesw_v, deg_sh.at[dst_v.at[half + b]],
                                     dsem, add=True))
      for d in sd + dd:
        d.wait()
    return 0
  lax.fori_loop(0, ngrp, _super, 0)
  plsc.subcore_barrier()

  off = c * N_PAD + s * RPT
  pltpu.sync_copy(acc_sh.at[pl.ds(s * RPT, RPT)], acc_out.at[pl.ds(off, RPT)])
  if with_deg:
    w = DEG_W // NS
    pltpu.sync_copy(deg_sh.at[pl.ds(s * w, w)],
                    deg_out.at[pl.ds(c * DEG_W + s * w, w)])


def _make_sc_kernel(with_deg):
  mesh = plsc.VectorSubcoreMesh(core_axis_name="c", subcore_axis_name="s")
  out_type = [jax.ShapeDtypeStruct((NC * N_PAD, H), jnp.float32)]
  scratch = [
      pltpu.VMEM((2 * GRP, CHUNK), jnp.int32),
      pltpu.VMEM((2 * GRP, CHUNK), jnp.int32),
  ]
  scratch += [pltpu.VMEM((CHUNK, H), jnp.float32) for _ in range(GRP)]
  if with_deg:
    out_type.append(jax.ShapeDtypeStruct((NC * DEG_W,), jnp.float32))
    scratch += [
        pltpu.VMEM((CHUNK,), jnp.float32),
        pltpu.VMEM((DEG_W // NS,), jnp.float32),
    ]
  scratch.append(pltpu.VMEM_SHARED((N_PAD, H), jnp.float32))
  if with_deg:
    scratch.append(pltpu.VMEM_SHARED((DEG_W,), jnp.float32))
  scratch += [pltpu.SemaphoreType.DMA for _ in range(2 * GRP)]
  if with_deg:
    scratch.append(pltpu.SemaphoreType.DMA)
  return pl.kernel(
      functools.partial(_sc_edge_body, with_deg),
      out_type=out_type, mesh=mesh, scratch_types=scratch,
      compiler_params=pltpu.CompilerParams(needs_layout_passes=False,
                                          use_tc_tiling_on_sc=False),
      name="sage_edge_agg_deg" if with_deg else "sage_edge_agg")


_sc_agg_deg = _make_sc_kernel(True)
_sc_agg = _make_sc_kernel(False)


BN = 1024


def _inv_deg(deg_ref):
  d = deg_ref[...]
  return 1.0 / jnp.maximum(d[0] + d[1], 1.0)


def _combine(acc_ref, deg_ref, y_ref):
  a = acc_ref[...]
  agg = a[0] + a[1]
  return jnp.maximum(agg * _inv_deg(deg_ref) + y_ref[...][:, H:], 0.0)


def _tc_pre_body(x_ref, w_ref, b_ref, y_ref):
  y_ref[...] = jnp.dot(x_ref[...], w_ref[...], **_DOT) + b_ref[...]


def _tc_mid_body(acc_ref, deg_ref, y_ref, w_ref, b_ref, yout_ref):
  h = _combine(acc_ref, deg_ref, y_ref)
  yout_ref[...] = jnp.dot(h, w_ref[...], **_DOT) + b_ref[...]


def _tc_fin_body(acc_ref, deg_ref, y_ref, lng_ref, lnb_ref, batch_ref,
                 u_ref, wpu_ref, wph_ref, bp_ref, wo_ref, bo_ref, out_ref):
  h = _combine(acc_ref, deg_ref, y_ref)
  mu = jnp.mean(h, axis=-1, keepdims=True)
  hc = h - mu
  var = jnp.mean(hc * hc, axis=-1, keepdims=True)
  hn = hc * lax.rsqrt(var + 1e-5) * lng_ref[...] + lnb_ref[...]
  gid = lax.broadcasted_iota(jnp.int32, (BN, G), 1)
  oneh = (batch_ref[...] == gid).astype(jnp.float32)
  up = jnp.dot(u_ref[...], wpu_ref[...], **_DOT)
  z = jnp.maximum(
      jnp.dot(hn, wph_ref[...], **_DOT) + jnp.dot(oneh, up, **_DOT)
      + bp_ref[...], 0.0)
  out_ref[...] = jnp.dot(z, wo_ref[...], **_DOT) + bo_ref[...]


def _row_spec(width):
  return pl.BlockSpec((BN, width), lambda i: (i, 0))


def _full_spec(shape):
  return pl.BlockSpec(shape, lambda i: tuple(0 for _ in shape))


def _acc_specs():
  return [pl.BlockSpec((2, BN, H), lambda i: (0, i, 0)),
          pl.BlockSpec((2, BN, 1), lambda i: (0, i, 0))]


_GRID = (10,)

_tc_pre = pl.pallas_call(
    _tc_pre_body,
    grid=_GRID,
    in_specs=[_row_spec(DIN), _full_spec((DIN, 2 * H)), _full_spec((1, 2 * H))],
    out_specs=_row_spec(2 * H),
    out_shape=jax.ShapeDtypeStruct((N, 2 * H), jnp.float32),
)

_tc_mid = pl.pallas_call(
    _tc_mid_body,
    grid=_GRID,
    in_specs=_acc_specs() + [_row_spec(2 * H), _full_spec((H, 2 * H)),
                             _full_spec((1, 2 * H))],
    out_specs=_row_spec(2 * H),
    out_shape=jax.ShapeDtypeStruct((N, 2 * H), jnp.float32),
)

_tc_fin = pl.pallas_call(
    _tc_fin_body,
    grid=_GRID,
    in_specs=_acc_specs() + [
        _row_spec(2 * H), _full_spec((1, H)), _full_spec((1, H)),
        _row_spec(1),
        _full_spec((G, 8)), _full_spec((8, H)),
        _full_spec((H, H)), _full_spec((1, H)),
        _full_spec((H, 2)), _full_spec((1, 2)),
    ],
    out_specs=_row_spec(2),
    out_shape=jax.ShapeDtypeStruct((N, 2), jnp.float32),
)


def kernel(x, edge_index, batch, u, Wl0, bl0, Wr0, Wl1, bl1, Wr1,
           Wl2, bl2, Wr2, ln_g, ln_b, Wp, bp, Wo, bo):
  pad = E_PAD - E
  src = jnp.concatenate([edge_index[0] * 2, jnp.zeros((pad,), jnp.int32)])
  dst = jnp.concatenate([edge_index[1], jnp.full((pad,), N, jnp.int32)])
  src2 = src.reshape(E_PAD // CHUNK, CHUNK)
  dst2 = dst.reshape(E_PAD // CHUNK, CHUNK)

  zH = jnp.zeros((H,), jnp.float32)
  w0 = jnp.concatenate([Wl0.T, Wr0.T], axis=1)
  w1 = jnp.concatenate([Wl1.T, Wr1.T], axis=1)
  w2 = jnp.concatenate([Wl2.T, Wr2.T], axis=1)
  b0 = jnp.concatenate([zH, bl0]).reshape(1, 2 * H)
  b1 = jnp.concatenate([zH, bl1]).reshape(1, 2 * H)
  b2 = jnp.concatenate([zH, bl2]).reshape(1, 2 * H)
  u_p = jnp.pad(u, ((0, 0), (0, 8 - GC)))
  wpu = jnp.pad(Wp[:, H:].T, ((0, 8 - GC), (0, 0)))
  wph = Wp[:, :H].T
  row = lambda v: v.reshape(1, -1)
  batch2 = batch.reshape(N, 1)

  y0 = _tc_pre(x, w0, b0)
  acc0, deg = _sc_agg_deg(y0.reshape(2 * N, H), src2, dst2)
  acc0 = acc0.reshape(NC, N_PAD, H)
  deg = deg.reshape(NC, DEG_W, 1)

  y1 = _tc_mid(acc0, deg, y0, w1, b1)
  acc1 = _sc_agg(y1.reshape(2 * N, H), src2, dst2)[0].reshape(NC, N_PAD, H)
  y2 = _tc_mid(acc1, deg, y1, w2, b2)
  acc2 = _sc_agg(y2.reshape(2 * N, H), src2, dst2)[0].reshape(NC, N_PAD, H)

  return _tc_fin(acc2, deg, y2, row(ln_g), row(ln_b), batch2,
                 u_p, wpu, wph, row(bp), Wo.T, row(bo))


# speedup vs baseline: 1.4276x; 1.0008x over previous
import functools

import jax
import jax.numpy as jnp
from jax import lax
from jax.experimental import pallas as pl
from jax.experimental.pallas import tpu as pltpu
from jax.experimental.pallas import tpu_sc as plsc

N = 10000
E = 320000
DIN = 128
H = 64
G = 16
GC = 3

NC = 2
NS = 16
CHUNK = 128
GRP = 8
CPT0 = 144
CPT1 = 16
E_PAD = NS * (CPT0 + CPT1) * CHUNK
RPT = 632
N_PAD = NS * RPT
DEG_W = 16384

_DOT = dict(precision=lax.Precision.HIGHEST, preferred_element_type=jnp.float32)


def _sc_edge_body(with_deg, *refs):
  if with_deg:
    (y_hbm, src_hbm, dst_hbm, acc_out, deg_out, src_v, dst_v,
     *rest) = refs
    rows = rest[:GRP]
    onesw_v, zw_v, acc_sh, deg_sh = rest[GRP:GRP + 4]
    gsem = rest[GRP + 4:2 * GRP + 4]
    ssem = rest[2 * GRP + 4:3 * GRP + 4]
    dsem = rest[3 * GRP + 4]
  else:
    (y_hbm, src_hbm, dst_hbm, acc_out, src_v, dst_v, *rest) = refs
    rows = rest[:GRP]
    acc_sh = rest[GRP]
    gsem = rest[GRP + 1:2 * GRP + 1]
    ssem = rest[2 * GRP + 1:3 * GRP + 1]
  c = lax.axis_index("c")
  s = lax.axis_index("s")
  cbase = jnp.where(c == 0, s * CPT0, NS * CPT0 + s * CPT1)
  ngrp = jnp.where(c == 0, CPT0 // (2 * GRP), CPT1 // (2 * GRP))
  zeros16 = jnp.zeros((16,), jnp.float32)
  ones16 = jnp.ones((16,), jnp.float32)

  def _z(i, _):
    rows[0][i >> 2, pl.ds((i & 3) * 16, 16)] = zeros16
    return 0
  lax.fori_loop(0, CHUNK * 4, _z, 0)
  for k in range(RPT // CHUNK):
    pltpu.sync_copy(rows[0], acc_sh.at[pl.ds(s * RPT + k * CHUNK, CHUNK)])
  rem = RPT - (RPT // CHUNK) * CHUNK
  pltpu.sync_copy(rows[0].at[pl.ds(0, rem)],
                  acc_sh.at[pl.ds(s * RPT + RPT - rem, rem)])
  if with_deg:
    for g in range(CHUNK // 16):
      onesw_v[pl.ds(g * 16, 16)] = ones16
    def _zd(i, _):
      zw_v[pl.ds(i * 16, 16)] = zeros16
      return 0
    lax.fori_loop(0, DEG_W // NS // 16, _zd, 0)
    pltpu.sync_copy(zw_v, deg_sh.at[pl.ds(s * (DEG_W // NS), DEG_W // NS)])
  plsc.subcore_barrier()

  def _super(u, _):
    pltpu.sync_copy(src_hbm.at[pl.ds(cbase + u * 2 * GRP, 2 * GRP)], src_v)
    pltpu.sync_copy(dst_hbm.at[pl.ds(cbase + u * 2 * GRP, 2 * GRP)], dst_v)
    for half in (0, GRP):
      gd = [pltpu.async_copy(y_hbm.at[src_v.at[half + b]], rows[b], gsem[b])
            for b in range(GRP)]
      sd = []
      dd = []
      for b in range(GRP):
        gd[b].wait()
        sd.append(pltpu.async_copy(rows[b], acc_sh.at[dst_v.at[half + b]],
                                   ssem[b], add=True))
        if with_deg:
          dd.append(pltpu.async_copy(onesw_v, deg_sh.at[dst_v.at[half + b]],
                                     dsem, add=True))
      for d in sd + dd:
        d.wait()
    return 0
  lax.fori_loop(0, ngrp, _super, 0)
  plsc.subcore_barrier()

  off = c * N_PAD + s * RPT
  pltpu.sync_copy(acc_sh.at[pl.ds(s * RPT, RPT)], acc_out.at[pl.ds(off, RPT)])
  if with_deg:
    w = DEG_W // NS
    pltpu.sync_copy(deg_sh.at[pl.ds(s * w, w)],
                    deg_out.at[pl.ds(c * DEG_W + s * w, w)])


def _make_sc_kernel(with_deg):
  mesh = plsc.VectorSubcoreMesh(core_axis_name="c", subcore_axis_name="s")
  out_type = [jax.ShapeDtypeStruct((NC * N_PAD, H), jnp.float32)]
  scratch = [
      pltpu.VMEM((2 * GRP, CHUNK), jnp.int32),
      pltpu.VMEM((2 * GRP, CHUNK), jnp.int32),
  ]
  scratch += [pltpu.VMEM((CHUNK, H), jnp.float32) for _ in range(GRP)]
  if with_deg:
    out_type.append(jax.ShapeDtypeStruct((NC * DEG_W,), jnp.float32))
    scratch += [
        pltpu.VMEM((CHUNK,), jnp.float32),
        pltpu.VMEM((DEG_W // NS,), jnp.float32),
    ]
  scratch.append(pltpu.VMEM_SHARED((N_PAD, H), jnp.float32))
  if with_deg:
    scratch.append(pltpu.VMEM_SHARED((DEG_W,), jnp.float32))
  scratch += [pltpu.SemaphoreType.DMA for _ in range(2 * GRP)]
  if with_deg:
    scratch.append(pltpu.SemaphoreType.DMA)
  return pl.kernel(
      functools.partial(_sc_edge_body, with_deg),
      out_type=out_type, mesh=mesh, scratch_types=scratch,
      compiler_params=pltpu.CompilerParams(needs_layout_passes=False,
                                          use_tc_tiling_on_sc=False),
      name="sage_edge_agg_deg" if with_deg else "sage_edge_agg")


_sc_agg_deg = _make_sc_kernel(True)
_sc_agg = _make_sc_kernel(False)


BN = 1024


def _inv_deg(deg_ref):
  d = deg_ref[...]
  return 1.0 / jnp.maximum(d[0] + d[1], 1.0)


def _combine(acc_ref, deg_ref, y_ref):
  a = acc_ref[...]
  agg = a[0] + a[1]
  return jnp.maximum(agg * _inv_deg(deg_ref) + y_ref[...][:, H:], 0.0)


def _tc_pre_body(x_ref, w_ref, b_ref, y_ref):
  y_ref[...] = jnp.dot(x_ref[...], w_ref[...], **_DOT) + b_ref[...]


def _tc_mid_body(acc_ref, deg_ref, y_ref, w_ref, b_ref, yout_ref):
  h = _combine(acc_ref, deg_ref, y_ref)
  yout_ref[...] = jnp.dot(h, w_ref[...], **_DOT) + b_ref[...]


def _tc_fin_body(acc_ref, deg_ref, y_ref, lng_ref, lnb_ref, batch_ref,
                 u_ref, wpu_ref, wph_ref, bp_ref, wo_ref, bo_ref, out_ref):
  h = _combine(acc_ref, deg_ref, y_ref)
  mu = jnp.mean(h, axis=-1, keepdims=True)
  hc = h - mu
  var = jnp.mean(hc * hc, axis=-1, keepdims=True)
  hn = hc * lax.rsqrt(var + 1e-5) * lng_ref[...] + lnb_ref[...]
  gid = lax.broadcasted_iota(jnp.int32, (BN, G), 1)
  oneh = (batch_ref[...] == gid).astype(jnp.float32)
  up = jnp.dot(u_ref[...], wpu_ref[...], **_DOT)
  z = jnp.maximum(
      jnp.dot(hn, wph_ref[...], **_DOT) + jnp.dot(oneh, up, **_DOT)
      + bp_ref[...], 0.0)
  out_ref[...] = jnp.dot(z, wo_ref[...], **_DOT) + bo_ref[...]


def _row_spec(width):
  return pl.BlockSpec((BN, width), lambda i: (i, 0))


def _full_spec(shape):
  return pl.BlockSpec(shape, lambda i: tuple(0 for _ in shape))


def _acc_specs():
  return [pl.BlockSpec((2, BN, H), lambda i: (0, i, 0)),
          pl.BlockSpec((2, BN, 1), lambda i: (0, i, 0))]


_GRID = (10,)

_tc_pre = pl.pallas_call(
    _tc_pre_body,
    grid=_GRID,
    in_specs=[_row_spec(DIN), _full_spec((DIN, 2 * H)), _full_spec((1, 2 * H))],
    out_specs=_row_spec(2 * H),
    out_shape=jax.ShapeDtypeStruct((N, 2 * H), jnp.float32),
)

_tc_mid = pl.pallas_call(
    _tc_mid_body,
    grid=_GRID,
    in_specs=_acc_specs() + [_row_spec(2 * H), _full_spec((H, 2 * H)),
                             _full_spec((1, 2 * H))],
    out_specs=_row_spec(2 * H),
    out_shape=jax.ShapeDtypeStruct((N, 2 * H), jnp.float32),
)

_tc_fin = pl.pallas_call(
    _tc_fin_body,
    grid=_GRID,
    in_specs=_acc_specs() + [
        _row_spec(2 * H), _full_spec((1, H)), _full_spec((1, H)),
        _row_spec(1),
        _full_spec((G, 8)), _full_spec((8, H)),
        _full_spec((H, H)), _full_spec((1, H)),
        _full_spec((H, 2)), _full_spec((1, 2)),
    ],
    out_specs=_row_spec(2),
    out_shape=jax.ShapeDtypeStruct((N, 2), jnp.float32),
)


def kernel(x, edge_index, batch, u, Wl0, bl0, Wr0, Wl1, bl1, Wr1,
           Wl2, bl2, Wr2, ln_g, ln_b, Wp, bp, Wo, bo):
  pad = E_PAD - E
  src = jnp.concatenate([edge_index[0] * 2, jnp.zeros((pad,), jnp.int32)])
  dst = jnp.concatenate([edge_index[1], jnp.full((pad,), N, jnp.int32)])
  src2 = src.reshape(E_PAD // CHUNK, CHUNK)
  dst2 = dst.reshape(E_PAD // CHUNK, CHUNK)

  zH = jnp.zeros((H,), jnp.float32)
  w0 = jnp.concatenate([Wl0.T, Wr0.T], axis=1)
  w1 = jnp.concatenate([Wl1.T, Wr1.T], axis=1)
  w2 = jnp.concatenate([Wl2.T, Wr2.T], axis=1)
  b0 = jnp.concatenate([zH, bl0]).reshape(1, 2 * H)
  b1 = jnp.concatenate([zH, bl1]).reshape(1, 2 * H)
  b2 = jnp.concatenate([zH, bl2]).reshape(1, 2 * H)
  u_p = jnp.pad(u, ((0, 0), (0, 8 - GC)))
  wpu = jnp.pad(Wp[:, H:].T, ((0, 8 - GC), (0, 0)))
  wph = Wp[:, :H].T
  row = lambda v: v.reshape(1, -1)
  batch2 = batch.reshape(N, 1)

  y0 = _tc_pre(x, w0, b0)
  acc0, deg = _sc_agg_deg(y0.reshape(2 * N, H), src2, dst2)
  acc0 = acc0.reshape(NC, N_PAD, H)
  deg = deg.reshape(NC, DEG_W, 1)

  y1 = _tc_mid(acc0, deg, y0, w1, b1)
  acc1 = _sc_agg(y1.reshape(2 * N, H), src2, dst2)[0].reshape(NC, N_PAD, H)
  y2 = _tc_mid(acc1, deg, y1, w2, b2)
  acc2 = _sc_agg(y2.reshape(2 * N, H), src2, dst2)[0].reshape(NC, N_PAD, H)

  return _tc_fin(acc2, deg, y2, row(ln_g), row(ln_b), batch2,
                 u_p, wpu, wph, row(bp), Wo.T, row(bo))


# speedup vs baseline: 1.5431x; 1.0809x over previous
import functools

import jax
import jax.numpy as jnp
from jax import lax
from jax.experimental import pallas as pl
from jax.experimental.pallas import tpu as pltpu
from jax.experimental.pallas import tpu_sc as plsc

N = 10000
E = 320000
DIN = 128
H = 64
G = 16
GC = 3

NC = 2
NS = 16
CHUNK = 128
GRP = 8
CPT0 = 144
CPT1 = 16
E_PAD = NS * (CPT0 + CPT1) * CHUNK
RPT = 632
N_PAD = NS * RPT
DEG_W = 16384

_DOT = dict(precision=lax.Precision.HIGHEST, preferred_element_type=jnp.float32)


def _sc_edge_body(with_deg, *refs):
  if with_deg:
    (y_hbm, src_hbm, dst_hbm, acc_out, deg_out, src_v, dst_v,
     *rest) = refs
    rows = rest[:GRP]
    onesw_v, zw_v, acc_sh, deg_sh = rest[GRP:GRP + 4]
    gsem = rest[GRP + 4:2 * GRP + 4]
    ssem = rest[2 * GRP + 4:3 * GRP + 4]
    dsem = rest[3 * GRP + 4]
  else:
    (y_hbm, src_hbm, dst_hbm, acc_out, src_v, dst_v, *rest) = refs
    rows = rest[:GRP]
    acc_sh = rest[GRP]
    gsem = rest[GRP + 1:2 * GRP + 1]
    ssem = rest[2 * GRP + 1:3 * GRP + 1]
  c = lax.axis_index("c")
  s = lax.axis_index("s")
  cbase = jnp.where(c == 0, s * CPT0, NS * CPT0 + s * CPT1)
  ngrp = jnp.where(c == 0, CPT0 // (2 * GRP), CPT1 // (2 * GRP))
  zeros16 = jnp.zeros((16,), jnp.float32)
  ones16 = jnp.ones((16,), jnp.float32)

  def _z(i, _):
    rows[0][i >> 2, pl.ds((i & 3) * 16, 16)] = zeros16
    return 0
  lax.fori_loop(0, CHUNK * 4, _z, 0)
  for k in range(RPT // CHUNK):
    pltpu.sync_copy(rows[0], acc_sh.at[pl.ds(s * RPT + k * CHUNK, CHUNK)])
  rem = RPT - (RPT // CHUNK) * CHUNK
  pltpu.sync_copy(rows[0].at[pl.ds(0, rem)],
                  acc_sh.at[pl.ds(s * RPT + RPT - rem, rem)])
  if with_deg:
    for g in range(CHUNK // 16):
      onesw_v[pl.ds(g * 16, 16)] = ones16
    def _zd(i, _):
      zw_v[pl.ds(i * 16, 16)] = zeros16
      return 0
    lax.fori_loop(0, DEG_W // NS // 16, _zd, 0)
    pltpu.sync_copy(zw_v, deg_sh.at[pl.ds(s * (DEG_W // NS), DEG_W // NS)])
  plsc.subcore_barrier()

  def _super(u, _):
    pltpu.sync_copy(src_hbm.at[pl.ds(cbase + u * 2 * GRP, 2 * GRP)], src_v)
    pltpu.sync_copy(dst_hbm.at[pl.ds(cbase + u * 2 * GRP, 2 * GRP)], dst_v)
    for half in (0, GRP):
      gd = [pltpu.async_copy(y_hbm.at[src_v.at[half + b]], rows[b], gsem[b])
            for b in range(GRP)]
      sd = []
      dd = []
      for b in range(GRP):
        gd[b].wait()
        sd.append(pltpu.async_copy(rows[b], acc_sh.at[dst_v.at[half + b]],
                                   ssem[b], add=True))
        if with_deg:
          dd.append(pltpu.async_copy(onesw_v, deg_sh.at[dst_v.at[half + b]],
                                     dsem, add=True))
      for d in sd + dd:
        d.wait()
    return 0
  lax.fori_loop(0, ngrp, _super, 0)
  plsc.subcore_barrier()

  off = c * N_PAD + s * RPT
  pltpu.sync_copy(acc_sh.at[pl.ds(s * RPT, RPT)], acc_out.at[pl.ds(off, RPT)])
  if with_deg:
    w = DEG_W // NS
    pltpu.sync_copy(deg_sh.at[pl.ds(s * w, w)],
                    deg_out.at[pl.ds(c * DEG_W + s * w, w)])


def _make_sc_kernel(with_deg):
  mesh = plsc.VectorSubcoreMesh(core_axis_name="c", subcore_axis_name="s")
  out_type = [jax.ShapeDtypeStruct((NC * N_PAD, H), jnp.float32)]
  scratch = [
      pltpu.VMEM((2 * GRP, CHUNK), jnp.int32),
      pltpu.VMEM((2 * GRP, CHUNK), jnp.int32),
  ]
  scratch += [pltpu.VMEM((CHUNK, H), jnp.float32) for _ in range(GRP)]
  if with_deg:
    out_type.append(jax.ShapeDtypeStruct((NC * DEG_W,), jnp.float32))
    scratch += [
        pltpu.VMEM((CHUNK,), jnp.float32),
        pltpu.VMEM((DEG_W // NS,), jnp.float32),
    ]
  scratch.append(pltpu.VMEM_SHARED((N_PAD, H), jnp.float32))
  if with_deg:
    scratch.append(pltpu.VMEM_SHARED((DEG_W,), jnp.float32))
  scratch += [pltpu.SemaphoreType.DMA for _ in range(2 * GRP)]
  if with_deg:
    scratch.append(pltpu.SemaphoreType.DMA)
  return pl.kernel(
      functools.partial(_sc_edge_body, with_deg),
      out_type=out_type, mesh=mesh, scratch_types=scratch,
      compiler_params=pltpu.CompilerParams(needs_layout_passes=False,
                                          use_tc_tiling_on_sc=False),
      name="sage_edge_agg_deg" if with_deg else "sage_edge_agg")


_sc_agg_deg = _make_sc_kernel(True)
_sc_agg = _make_sc_kernel(False)


BN = 1024


def _inv_deg(deg_ref):
  d = deg_ref[...]
  return 1.0 / jnp.maximum(d[0] + d[1], 1.0)


def _combine(acc_ref, deg_ref, r_ref):
  a = acc_ref[...]
  agg = a[0] + a[1]
  return jnp.maximum(agg * _inv_deg(deg_ref) + r_ref[...], 0.0)


def _tc_pre_body(x_ref, w_ref, b_ref, p_ref, r_ref):
  y = jnp.dot(x_ref[...], w_ref[...], **_DOT) + b_ref[...]
  p_ref[...] = y[:, :H]
  r_ref[...] = y[:, H:]


def _tc_mid_body(acc_ref, deg_ref, r_ref, w_ref, b_ref, p_ref, rout_ref):
  h = _combine(acc_ref, deg_ref, r_ref)
  y = jnp.dot(h, w_ref[...], **_DOT) + b_ref[...]
  p_ref[...] = y[:, :H]
  rout_ref[...] = y[:, H:]


def _tc_fin_body(acc_ref, deg_ref, y_ref, lng_ref, lnb_ref, batch_ref,
                 u_ref, wpu_ref, wph_ref, bp_ref, wo_ref, bo_ref, out_ref):
  h = _combine(acc_ref, deg_ref, y_ref)
  mu = jnp.mean(h, axis=-1, keepdims=True)
  hc = h - mu
  var = jnp.mean(hc * hc, axis=-1, keepdims=True)
  hn = hc * lax.rsqrt(var + 1e-5) * lng_ref[...] + lnb_ref[...]
  gid = lax.broadcasted_iota(jnp.int32, (BN, G), 1)
  oneh = (batch_ref[...] == gid).astype(jnp.float32)
  up = jnp.dot(u_ref[...], wpu_ref[...], **_DOT)
  z = jnp.maximum(
      jnp.dot(hn, wph_ref[...], **_DOT) + jnp.dot(oneh, up, **_DOT)
      + bp_ref[...], 0.0)
  out_ref[...] = jnp.dot(z, wo_ref[...], **_DOT) + bo_ref[...]


def _row_spec(width):
  return pl.BlockSpec((BN, width), lambda i: (i, 0))


def _full_spec(shape):
  return pl.BlockSpec(shape, lambda i: tuple(0 for _ in shape))


def _acc_specs():
  return [pl.BlockSpec((2, BN, H), lambda i: (0, i, 0)),
          pl.BlockSpec((2, BN, 1), lambda i: (0, i, 0))]


_GRID = (10,)

_tc_pre = pl.pallas_call(
    _tc_pre_body,
    grid=_GRID,
    in_specs=[_row_spec(DIN), _full_spec((DIN, 2 * H)), _full_spec((1, 2 * H))],
    out_specs=[_row_spec(H), _row_spec(H)],
    out_shape=[jax.ShapeDtypeStruct((N, H), jnp.float32)] * 2,
)

_tc_mid = pl.pallas_call(
    _tc_mid_body,
    grid=_GRID,
    in_specs=_acc_specs() + [_row_spec(H), _full_spec((H, 2 * H)),
                             _full_spec((1, 2 * H))],
    out_specs=[_row_spec(H), _row_spec(H)],
    out_shape=[jax.ShapeDtypeStruct((N, H), jnp.float32)] * 2,
)

_tc_fin = pl.pallas_call(
    _tc_fin_body,
    grid=_GRID,
    in_specs=_acc_specs() + [
        _row_spec(H), _full_spec((1, H)), _full_spec((1, H)),
        _row_spec(1),
        _full_spec((G, 8)), _full_spec((8, H)),
        _full_spec((H, H)), _full_spec((1, H)),
        _full_spec((H, 2)), _full_spec((1, 2)),
    ],
    out_specs=_row_spec(2),
    out_shape=jax.ShapeDtypeStruct((N, 2), jnp.float32),
)


def kernel(x, edge_index, batch, u, Wl0, bl0, Wr0, Wl1, bl1, Wr1,
           Wl2, bl2, Wr2, ln_g, ln_b, Wp, bp, Wo, bo):
  pad = E_PAD - E
  src = jnp.concatenate([edge_index[0], jnp.zeros((pad,), jnp.int32)])
  dst = jnp.concatenate([edge_index[1], jnp.full((pad,), N, jnp.int32)])
  src2 = src.reshape(E_PAD // CHUNK, CHUNK)
  dst2 = dst.reshape(E_PAD // CHUNK, CHUNK)

  zH = jnp.zeros((H,), jnp.float32)
  w0 = jnp.concatenate([Wl0.T, Wr0.T], axis=1)
  w1 = jnp.concatenate([Wl1.T, Wr1.T], axis=1)
  w2 = jnp.concatenate([Wl2.T, Wr2.T], axis=1)
  b0 = jnp.concatenate([zH, bl0]).reshape(1, 2 * H)
  b1 = jnp.concatenate([zH, bl1]).reshape(1, 2 * H)
  b2 = jnp.concatenate([zH, bl2]).reshape(1, 2 * H)
  u_p = jnp.pad(u, ((0, 0), (0, 8 - GC)))
  wpu = jnp.pad(Wp[:, H:].T, ((0, 8 - GC), (0, 0)))
  wph = Wp[:, :H].T
  row = lambda v: v.reshape(1, -1)
  batch2 = batch.reshape(N, 1)

  p0, r0 = _tc_pre(x, w0, b0)
  acc0, deg = _sc_agg_deg(p0, src2, dst2)
  acc0 = acc0.reshape(NC, N_PAD, H)
  deg = deg.reshape(NC, DEG_W, 1)

  p1, r1 = _tc_mid(acc0, deg, r0, w1, b1)
  acc1 = _sc_agg(p1, src2, dst2)[0].reshape(NC, N_PAD, H)
  p2, r2 = _tc_mid(acc1, deg, r1, w2, b2)
  acc2 = _sc_agg(p2, src2, dst2)[0].reshape(NC, N_PAD, H)

  return _tc_fin(acc2, deg, r2, row(ln_g), row(ln_b), batch2,
                 u_p, wpu, wph, row(bp), Wo.T, row(bo))
